# Initial kernel scaffold; baseline (speedup 1.0000x reference)
import jax
import jax.numpy as jnp
from jax.experimental import pallas as pl


def kernel(uR, g_logscale, logistic_noise):
    raise NotImplementedError("write your pallas kernel here")


# speedup vs baseline: 146.5054x; 146.5054x over previous
import functools

import jax
import jax.numpy as jnp
import numpy as np
from jax import lax
from jax.experimental import pallas as pl
from jax.experimental.pallas import tpu as pltpu
from jax.experimental.pallas import tpu_sc as plsc

N = 2048
DIM_U = 16
TEMPERATURE = 0.3
NPAIRS = N * (N - 1) // 2
PAD = 8
NOISE_LEN = 2_096_256
WIN = N + 16
RB = 128
NW = 32
ROWS_PER_W = N // NW


def _rank_kernel(o_col_ref, o_row_ref, rank_ref, cval_ref):
    i = pl.program_id(0)
    oa = o_col_ref[...]
    ob = o_row_ref[...]
    bidx = lax.broadcasted_iota(jnp.int32, (RB, N), 1)
    aidx = i * RB + lax.broadcasted_iota(jnp.int32, (RB, N), 0)
    less = (ob < oa) | ((ob == oa) & (bidx < aidx))
    r = jnp.sum(less.astype(jnp.int32), axis=1, keepdims=True)
    rank_ref[...] = r
    cval_ref[...] = ((r * (2 * N - 1 - r)) // 2) - r - 1 + PAD


@functools.cache
def _make_noise_permute():
    mesh = plsc.VectorSubcoreMesh(core_axis_name="c", subcore_axis_name="s")
    return functools.partial(
        pl.kernel,
        mesh=mesh,
        compiler_params=pltpu.CompilerParams(needs_layout_passes=False),
        out_type=jax.ShapeDtypeStruct((N, N), jnp.float32),
        scratch_types=[
            pltpu.VMEM((N,), jnp.int32),
            pltpu.VMEM((ROWS_PER_W + 16,), jnp.int32),
            pltpu.VMEM((WIN,), jnp.float32),
            pltpu.VMEM((N,), jnp.float32),
        ],
    )(_noise_permute_body)


def _noise_permute_body(rank_hbm, cval_hbm, noise_hbm, out_hbm, rank_v, cv_v, win_v, row_v):
    wid = lax.axis_index("s") * 2 + lax.axis_index("c")
    base = wid * ROWS_PER_W
    pltpu.sync_copy(rank_hbm, rank_v)
    pltpu.sync_copy(cval_hbm.at[pl.ds(base, ROWS_PER_W)],
                    cv_v.at[pl.ds(0, ROWS_PER_W)])

    def row_body(r, _):
        c = cv_v[pl.ds(r, 16)][0]
        al = pl.multiple_of(c & (-16), 16)
        delta = c - al
        pltpu.sync_copy(noise_hbm.at[pl.ds(al, WIN)], win_v)
        for k in range(N // 16):
            idx = rank_v[pl.ds(k * 16, 16)] + delta
            row_v[pl.ds(k * 16, 16)] = plsc.load_gather(win_v, [idx])
        pltpu.sync_copy(row_v, out_hbm.at[base + r])
        return 0

    lax.fori_loop(0, ROWS_PER_W, row_body, 0)


def _fuse_kernel(uR_ref, g_ref, nz_ref, rank_col_ref, rank_row_ref, out_ref):
    i = pl.program_id(0)
    U = uR_ref[...]
    X = uR_ref[pl.ds(i * RB, RB), :]
    G2 = lax.dot_general(X, U, (((1,), (1,)), ((), ())),
                         preferred_element_type=jnp.float32)
    rn_rows = jnp.sum(X * X, axis=1, keepdims=True)
    ones = jnp.ones((1, DIM_U), jnp.float32)
    rn_cols = lax.dot_general(ones, U * U, (((1,), (1,)), ((), ())),
                              preferred_element_type=jnp.float32)
    D = rn_rows + rn_cols - 2.0 * G2
    s = jnp.exp(g_ref[...])
    a = (-0.5 * D) / s
    c = -0.69314718056
    pos = jnp.clip(a, c, None)
    neg = jnp.clip(a, None, c)
    neg_val = neg - jnp.log(1.0 - jnp.exp(neg))
    pos_val = -jnp.log(jnp.clip(jnp.exp(-pos) - 1.0, 1e-20, None))
    logit = pos_val + neg_val
    x = (logit + nz_ref[...]) / TEMPERATURE
    sig = 1.0 / (1.0 + jnp.exp(-x))
    mask = rank_col_ref[...] < rank_row_ref[...]
    out_ref[...] = jnp.where(mask, sig, 0.0)


def _rank_call(o_col, o_row):
    return pl.pallas_call(
        _rank_kernel,
        grid=(N // RB,),
        in_specs=[
            pl.BlockSpec((RB, 1), lambda i: (i, 0)),
            pl.BlockSpec((1, N), lambda i: (0, 0)),
        ],
        out_specs=[
            pl.BlockSpec((RB, 1), lambda i: (i, 0)),
            pl.BlockSpec((RB, 1), lambda i: (i, 0)),
        ],
        out_shape=[
            jax.ShapeDtypeStruct((N, 1), jnp.int32),
            jax.ShapeDtypeStruct((N, 1), jnp.int32),
        ],
    )(o_col, o_row)


def _fuse_call(uR, g2d, nz, rank_col, rank_row):
    return pl.pallas_call(
        _fuse_kernel,
        grid=(N // RB,),
        in_specs=[
            pl.BlockSpec((N, DIM_U), lambda i: (0, 0)),
            pl.BlockSpec((1, 1), lambda i: (0, 0)),
            pl.BlockSpec((RB, N), lambda i: (i, 0)),
            pl.BlockSpec((RB, 1), lambda i: (i, 0)),
            pl.BlockSpec((1, N), lambda i: (0, 0)),
        ],
        out_specs=pl.BlockSpec((RB, N), lambda i: (i, 0)),
        out_shape=jax.ShapeDtypeStruct((N, N), jnp.float32),
    )(uR, g2d, nz, rank_col, rank_row)


def kernel(uR, g_logscale, logistic_noise):
    o = jnp.sum(jnp.log(0.5 + 0.5 * jax.scipy.special.erf(uR / np.sqrt(2.0))),
                axis=1, keepdims=True)
    o_row = o.reshape(1, N)
    rank2, cval2 = _rank_call(o, o_row)
    noise_p = jnp.pad(logistic_noise.astype(jnp.float32),
                      (PAD, NOISE_LEN - PAD - NPAIRS))
    nz = _make_noise_permute()(rank2.reshape(N), cval2.reshape(N), noise_p)
    g2d = g_logscale.reshape(1, 1).astype(jnp.float32)
    return _fuse_call(uR, g2d, nz, rank2, rank2.reshape(1, N))


# speedup vs baseline: 213.3464x; 1.4562x over previous
import functools

import jax
import jax.numpy as jnp
import numpy as np
from jax import lax
from jax.experimental import pallas as pl
from jax.experimental.pallas import tpu as pltpu
from jax.experimental.pallas import tpu_sc as plsc

N = 2048
DIM_U = 16
TEMPERATURE = 0.3
NPAIRS = N * (N - 1) // 2
WIN = N + 32
BUF = WIN + 16
RB = 128
NW = 32
ROWS_PER_W = N // NW


def _rank_kernel(o_col_ref, o_row_ref, rank_ref, cval_ref):
    i = pl.program_id(0)
    oa = o_col_ref[...]
    ob = o_row_ref[...]
    bidx = lax.broadcasted_iota(jnp.int32, (RB, N), 1)
    aidx = i * RB + lax.broadcasted_iota(jnp.int32, (RB, N), 0)
    less = (ob < oa) | ((ob == oa) & (bidx < aidx))
    r = jnp.sum(less.astype(jnp.int32), axis=1, keepdims=True)
    rank_ref[...] = r
    cval_ref[...] = ((r * (2 * N - 1 - r)) // 2) - r - 1


@functools.cache
def _make_noise_permute():
    mesh = plsc.VectorSubcoreMesh(core_axis_name="c", subcore_axis_name="s")
    return functools.partial(
        pl.kernel,
        mesh=mesh,
        compiler_params=pltpu.CompilerParams(needs_layout_passes=False),
        out_type=jax.ShapeDtypeStruct((N, N), jnp.float32),
        scratch_types=[
            pltpu.VMEM((N,), jnp.int32),
            pltpu.VMEM((ROWS_PER_W + 16,), jnp.int32),
            pltpu.VMEM((BUF,), jnp.float32),
            pltpu.VMEM((BUF,), jnp.float32),
            pltpu.VMEM((N,), jnp.float32),
            pltpu.VMEM((N,), jnp.float32),
            pltpu.SemaphoreType.DMA,
            pltpu.SemaphoreType.DMA,
            pltpu.SemaphoreType.DMA,
            pltpu.SemaphoreType.DMA,
        ],
    )(_noise_permute_body)


def _noise_permute_body(rank_hbm, cval_hbm, noise_hbm, out_hbm, rank_v, cv_v,
                        win0, win1, row0, row1, sin0, sin1, so0, so1):
    wid = lax.axis_index("s") * 2 + lax.axis_index("c")
    base = wid * ROWS_PER_W
    pltpu.sync_copy(rank_hbm, rank_v)
    pltpu.sync_copy(cval_hbm.at[pl.ds(base, ROWS_PER_W)],
                    cv_v.at[pl.ds(0, ROWS_PER_W)])

    def aligned_start(c):
        return jnp.clip(c & (-16), 0, NPAIRS - WIN)

    def win_start(r, wbuf, sem):
        c = cv_v[pl.ds(jnp.minimum(r, ROWS_PER_W - 1), 16)][0]
        al = pl.multiple_of(aligned_start(c), 16)
        pltpu.async_copy(noise_hbm.at[pl.ds(al, WIN)], wbuf.at[pl.ds(16, WIN)], sem)

    def win_wait(wbuf, sem):
        pltpu.make_async_copy(noise_hbm.at[pl.ds(0, WIN)],
                              wbuf.at[pl.ds(16, WIN)], sem).wait()

    def gather(r, wbuf, rbuf):
        c = cv_v[pl.ds(r, 16)][0]
        d16 = c - aligned_start(c) + 16
        for k in range(N // 16):
            idx = rank_v[pl.ds(k * 16, 16)] + d16
            rbuf[pl.ds(k * 16, 16)] = plsc.load_gather(wbuf, [idx])

    def out_start(r, rbuf, sem):
        pltpu.async_copy(rbuf, out_hbm.at[base + r], sem)

    def out_wait(rbuf, sem):
        pltpu.make_async_copy(noise_hbm.at[pl.ds(0, N)], rbuf, sem).wait()

    win_start(0, win0, sin0)

    def pair(p, _):
        r0 = 2 * p
        win_start(r0 + 1, win1, sin1)
        win_wait(win0, sin0)

        @pl.when(p > 0)
        def _():
            out_wait(row0, so0)

        gather(r0, win0, row0)
        out_start(r0, row0, so0)
        win_start(r0 + 2, win0, sin0)
        win_wait(win1, sin1)

        @pl.when(p > 0)
        def _():
            out_wait(row1, so1)

        gather(r0 + 1, win1, row1)
        out_start(r0 + 1, row1, so1)
        return 0

    lax.fori_loop(0, ROWS_PER_W // 2, pair, 0)
    win_wait(win0, sin0)
    out_wait(row0, so0)
    out_wait(row1, so1)


def _fuse_kernel(uR_ref, g_ref, nz_ref, rank_col_ref, rank_row_ref, out_ref):
    i = pl.program_id(0)
    U = uR_ref[...]
    X = uR_ref[pl.ds(i * RB, RB), :]
    G2 = lax.dot_general(X, U, (((1,), (1,)), ((), ())),
                         preferred_element_type=jnp.float32)
    rn_rows = jnp.sum(X * X, axis=1, keepdims=True)
    ones = jnp.ones((1, DIM_U), jnp.float32)
    rn_cols = lax.dot_general(ones, U * U, (((1,), (1,)), ((), ())),
                              preferred_element_type=jnp.float32)
    D = rn_rows + rn_cols - 2.0 * G2
    s = jnp.exp(g_ref[...])
    a = (-0.5 * D) / s
    c = -0.69314718056
    pos = jnp.clip(a, c, None)
    neg = jnp.clip(a, None, c)
    neg_val = neg - jnp.log(1.0 - jnp.exp(neg))
    pos_val = -jnp.log(jnp.clip(jnp.exp(-pos) - 1.0, 1e-20, None))
    logit = pos_val + neg_val
    x = (logit + nz_ref[...]) / TEMPERATURE
    sig = 1.0 / (1.0 + jnp.exp(-x))
    mask = rank_col_ref[...] < rank_row_ref[...]
    out_ref[...] = jnp.where(mask, sig, 0.0)


def _rank_call(o_col, o_row):
    return pl.pallas_call(
        _rank_kernel,
        grid=(N // RB,),
        in_specs=[
            pl.BlockSpec((RB, 1), lambda i: (i, 0)),
            pl.BlockSpec((1, N), lambda i: (0, 0)),
        ],
        out_specs=[
            pl.BlockSpec((RB, 1), lambda i: (i, 0)),
            pl.BlockSpec((RB, 1), lambda i: (i, 0)),
        ],
        out_shape=[
            jax.ShapeDtypeStruct((N, 1), jnp.int32),
            jax.ShapeDtypeStruct((N, 1), jnp.int32),
        ],
    )(o_col, o_row)


def _fuse_call(uR, g2d, nz, rank_col, rank_row):
    return pl.pallas_call(
        _fuse_kernel,
        grid=(N // RB,),
        in_specs=[
            pl.BlockSpec((N, DIM_U), lambda i: (0, 0)),
            pl.BlockSpec((1, 1), lambda i: (0, 0)),
            pl.BlockSpec((RB, N), lambda i: (i, 0)),
            pl.BlockSpec((RB, 1), lambda i: (i, 0)),
            pl.BlockSpec((1, N), lambda i: (0, 0)),
        ],
        out_specs=pl.BlockSpec((RB, N), lambda i: (i, 0)),
        out_shape=jax.ShapeDtypeStruct((N, N), jnp.float32),
    )(uR, g2d, nz, rank_col, rank_row)


def kernel(uR, g_logscale, logistic_noise):
    o = jnp.sum(jnp.log(0.5 + 0.5 * jax.scipy.special.erf(uR / np.sqrt(2.0))),
                axis=1, keepdims=True)
    o_row = o.reshape(1, N)
    rank2, cval2 = _rank_call(o, o_row)
    nz = _make_noise_permute()(rank2.reshape(N), cval2.reshape(N), logistic_noise)
    g2d = g_logscale.reshape(1, 1).astype(jnp.float32)
    return _fuse_call(uR, g2d, nz, rank2, rank2.reshape(1, N))
